# baseline (speedup 1.0000x reference)
import jax
import jax.numpy as jnp
from jax import lax
from jax.experimental import pallas as pl
from jax.experimental.pallas import tpu as pltpu, tpu_sc as plsc

_NC = 2
_NS = 16
_NW = _NC * _NS

_N = 2 * 256 * 256
_H = 4
_D = 64
_W = _H * _D
_RW = _N // _NW
_CH = 128
_NCHUNK = _RW // _CH


def _sc_body(tabk_hbm, tabv_hbm, idx_hbm, outk_hbm, outv_hbm,
             idx_v, bufk_v, bufv_v, sem):
    wid = lax.axis_index("s") * _NC + lax.axis_index("c")
    base = wid * _RW

    def chunk(o, carry):
        row0 = base + o * _CH
        pltpu.sync_copy(idx_hbm.at[pl.ds(row0, _CH)], idx_v)
        pltpu.async_copy(tabk_hbm.at[idx_v], bufk_v, sem).wait()
        pltpu.async_copy(tabv_hbm.at[idx_v], bufv_v, sem).wait()
        pltpu.sync_copy(bufk_v, outk_hbm.at[pl.ds(row0, _CH), 0])
        pltpu.sync_copy(bufk_v, outk_hbm.at[pl.ds(row0, _CH), 1])
        pltpu.sync_copy(bufv_v, outv_hbm.at[pl.ds(row0, _CH), 0])
        pltpu.sync_copy(bufv_v, outv_hbm.at[pl.ds(row0, _CH), 1])
        return carry

    lax.fori_loop(0, _NCHUNK, chunk, 0)


def kernel(inputs, brother_table, relation_type, num_heads):
    del relation_type, num_heads
    idx = inputs.reshape(-1).astype(jnp.int32)
    tab_k2 = jnp.tile(brother_table[:, :_D], (1, 2))
    tab_v2 = jnp.tile(brother_table[:, _D:], (1, 2))

    mesh = plsc.VectorSubcoreMesh(core_axis_name="c", subcore_axis_name="s")
    f = pl.kernel(
        _sc_body,
        out_type=(
            jax.ShapeDtypeStruct((_N, 2, 2 * _D), jnp.float32),
            jax.ShapeDtypeStruct((_N, 2, 2 * _D), jnp.float32),
        ),
        mesh=mesh,
        scratch_types=[
            pltpu.VMEM((_CH,), jnp.int32),
            pltpu.VMEM((_CH, 2 * _D), jnp.float32),
            pltpu.VMEM((_CH, 2 * _D), jnp.float32),
            pltpu.SemaphoreType.DMA,
        ],
    )
    outk, outv = f(tab_k2, tab_v2, idx)
    out_shape = (inputs.shape[0] * _H, inputs.shape[1], inputs.shape[2], _D)
    return outk.reshape(out_shape), outv.reshape(out_shape)


# speedup vs baseline: 1.0379x; 1.0379x over previous
import jax
import jax.numpy as jnp
from jax import lax
from jax.experimental import pallas as pl
from jax.experimental.pallas import tpu as pltpu, tpu_sc as plsc

_NC = 2
_NS = 16
_NW = _NC * _NS

_N = 2 * 256 * 256
_H = 4
_D = 64
_W = _H * _D
_RW = _N // _NW
_CH = 64
_NSTEP = _RW // (2 * _CH)


def _sc_body(tabk_hbm, tabv_hbm, idx_hbm, outk_hbm, outv_hbm,
             idx_v, bk0, bv0, bk1, bv1, gs0, gs1, ws0, ws1):
    wid = lax.axis_index("s") * _NC + lax.axis_index("c")
    base = wid * _RW
    pltpu.sync_copy(idx_hbm.at[pl.ds(base, _RW)], idx_v)

    def step(t, carry):
        oa = 2 * t * _CH
        ob = oa + _CH
        rowa = base + oa
        rowb = base + ob
        ia = idx_v.at[pl.ds(oa, _CH)]
        ib = idx_v.at[pl.ds(ob, _CH)]

        @pl.when(t != 0)
        def _():
            pltpu.make_async_copy(bk0, outk_hbm.at[pl.ds(rowa, _CH)], ws0).wait()
            pltpu.make_async_copy(bv0, outv_hbm.at[pl.ds(rowa, _CH)], ws0).wait()

        pltpu.async_copy(tabk_hbm.at[ia], bk0, gs0)
        pltpu.async_copy(tabv_hbm.at[ia], bv0, gs0)

        @pl.when(t != 0)
        def _():
            pltpu.make_async_copy(bk1, outk_hbm.at[pl.ds(rowb, _CH)], ws1).wait()
            pltpu.make_async_copy(bv1, outv_hbm.at[pl.ds(rowb, _CH)], ws1).wait()

        pltpu.async_copy(tabk_hbm.at[ib], bk1, gs1)
        pltpu.async_copy(tabv_hbm.at[ib], bv1, gs1)

        pltpu.make_async_copy(tabk_hbm.at[ia], bk0, gs0).wait()
        pltpu.make_async_copy(tabv_hbm.at[ia], bv0, gs0).wait()
        pltpu.async_copy(bk0, outk_hbm.at[pl.ds(rowa, _CH)], ws0)
        pltpu.async_copy(bv0, outv_hbm.at[pl.ds(rowa, _CH)], ws0)

        pltpu.make_async_copy(tabk_hbm.at[ib], bk1, gs1).wait()
        pltpu.make_async_copy(tabv_hbm.at[ib], bv1, gs1).wait()
        pltpu.async_copy(bk1, outk_hbm.at[pl.ds(rowb, _CH)], ws1)
        pltpu.async_copy(bv1, outv_hbm.at[pl.ds(rowb, _CH)], ws1)
        return carry

    lax.fori_loop(0, _NSTEP, step, 0)

    pltpu.make_async_copy(bk0, outk_hbm.at[pl.ds(base, _CH)], ws0).wait()
    pltpu.make_async_copy(bv0, outv_hbm.at[pl.ds(base, _CH)], ws0).wait()
    pltpu.make_async_copy(bk1, outk_hbm.at[pl.ds(base, _CH)], ws1).wait()
    pltpu.make_async_copy(bv1, outv_hbm.at[pl.ds(base, _CH)], ws1).wait()


def kernel(inputs, brother_table, relation_type, num_heads):
    del relation_type, num_heads
    idx = inputs.reshape(-1).astype(jnp.int32)
    tab_k4 = jnp.tile(brother_table[:, :_D], (1, _H))
    tab_v4 = jnp.tile(brother_table[:, _D:], (1, _H))

    mesh = plsc.VectorSubcoreMesh(core_axis_name="c", subcore_axis_name="s")
    f = pl.kernel(
        _sc_body,
        out_type=(
            jax.ShapeDtypeStruct((_N, _W), jnp.float32),
            jax.ShapeDtypeStruct((_N, _W), jnp.float32),
        ),
        mesh=mesh,
        scratch_types=[
            pltpu.VMEM((_RW,), jnp.int32),
            pltpu.VMEM((_CH, _W), jnp.float32),
            pltpu.VMEM((_CH, _W), jnp.float32),
            pltpu.VMEM((_CH, _W), jnp.float32),
            pltpu.VMEM((_CH, _W), jnp.float32),
            pltpu.SemaphoreType.DMA,
            pltpu.SemaphoreType.DMA,
            pltpu.SemaphoreType.DMA,
            pltpu.SemaphoreType.DMA,
        ],
    )
    outk, outv = f(tab_k4, tab_v4, idx)
    out_shape = (inputs.shape[0] * _H, inputs.shape[1], inputs.shape[2], _D)
    return outk.reshape(out_shape), outv.reshape(out_shape)
